# baseline (speedup 1.0000x reference)
import jax
from jax.experimental import pallas as pl

_GRID = 25


def _copy_body(ea_ref, x_ref, u_ref, eao_ref, xo_ref, uo_ref):
    eao_ref[...] = ea_ref[...]
    xo_ref[...] = x_ref[...]
    uo_ref[...] = u_ref[...]


def kernel(x, edge_index, edge_attr, u, batch, queries, num_props):
    ea_t = edge_attr.T
    n_ea = ea_t.shape[1] // _GRID
    n_x = x.shape[0] // _GRID
    eas = pl.BlockSpec((ea_t.shape[0], n_ea), lambda i: (0, i))
    xs = pl.BlockSpec((n_x, x.shape[1]), lambda i: (i, 0))
    us = pl.BlockSpec(u.shape, lambda i: (0, 0))
    outs = pl.pallas_call(
        _copy_body,
        grid=(_GRID,),
        out_shape=(
            jax.ShapeDtypeStruct(ea_t.shape, ea_t.dtype),
            jax.ShapeDtypeStruct(x.shape, x.dtype),
            jax.ShapeDtypeStruct(u.shape, u.dtype),
        ),
        in_specs=[eas, xs, us],
        out_specs=(eas, xs, us),
    )(ea_t, x, u)
    return (outs[1], outs[0].T, outs[2])


# speedup vs baseline: 1.5360x; 1.5360x over previous
import jax
from jax.experimental import pallas as pl

_GRID = 5


def _copy_body(ea_ref, x_ref, u_ref, eao_ref, xo_ref, uo_ref):
    eao_ref[...] = ea_ref[...]
    xo_ref[...] = x_ref[...]
    uo_ref[...] = u_ref[...]


def kernel(x, edge_index, edge_attr, u, batch, queries, num_props):
    ea_t = edge_attr.T
    n_ea = ea_t.shape[1] // _GRID
    n_x = x.shape[0] // _GRID
    eas = pl.BlockSpec((ea_t.shape[0], n_ea), lambda i: (0, i))
    xs = pl.BlockSpec((n_x, x.shape[1]), lambda i: (i, 0))
    us = pl.BlockSpec(u.shape, lambda i: (0, 0))
    outs = pl.pallas_call(
        _copy_body,
        grid=(_GRID,),
        out_shape=(
            jax.ShapeDtypeStruct(ea_t.shape, ea_t.dtype),
            jax.ShapeDtypeStruct(x.shape, x.dtype),
            jax.ShapeDtypeStruct(u.shape, u.dtype),
        ),
        in_specs=[eas, xs, us],
        out_specs=(eas, xs, us),
    )(ea_t, x, u)
    return (outs[1], outs[0].T, outs[2])


# speedup vs baseline: 1.6661x; 1.0847x over previous
import jax
from jax.experimental import pallas as pl

_GRID = 2


def _copy_body(ea_ref, x_ref, u_ref, eao_ref, xo_ref, uo_ref):
    eao_ref[...] = ea_ref[...]
    xo_ref[...] = x_ref[...]
    uo_ref[...] = u_ref[...]


def kernel(x, edge_index, edge_attr, u, batch, queries, num_props):
    ea_t = edge_attr.T
    n_ea = ea_t.shape[1] // _GRID
    n_x = x.shape[0] // _GRID
    eas = pl.BlockSpec((ea_t.shape[0], n_ea), lambda i: (0, i))
    xs = pl.BlockSpec((n_x, x.shape[1]), lambda i: (i, 0))
    us = pl.BlockSpec(u.shape, lambda i: (0, 0))
    outs = pl.pallas_call(
        _copy_body,
        grid=(_GRID,),
        out_shape=(
            jax.ShapeDtypeStruct(ea_t.shape, ea_t.dtype),
            jax.ShapeDtypeStruct(x.shape, x.dtype),
            jax.ShapeDtypeStruct(u.shape, u.dtype),
        ),
        in_specs=[eas, xs, us],
        out_specs=(eas, xs, us),
    )(ea_t, x, u)
    return (outs[1], outs[0].T, outs[2])
